# baseline (speedup 1.0000x reference)
import jax
import jax.numpy as jnp
from jax import lax
from jax.experimental import pallas as pl
from jax.experimental.pallas import tpu as pltpu
from jax.experimental.pallas import tpu_sc as plsc

N = 10000
D = 128
E = 320000
NC, NS = 2, 16
NW = NC * NS
B = 128
NBUF = 2
CH = ((-(-E // (NW * B)) + NBUF - 1) // NBUF) * NBUF
PH = CH // 2
EPAD = NW * CH * B
SB = 5
STRIPE = SB * B
NP = NS * STRIPE

_mesh = plsc.VectorSubcoreMesh(
    core_axis_name="c", subcore_axis_name="s", num_cores=NC, num_subcores=NS
)
_f32 = jnp.float32


def _zero_stripe(sh, buf, s):
    for k in range(SB):
        pltpu.sync_copy(buf, sh.at[pl.ds(s * STRIPE + k * B, B)])


def _copy_out_stripe(sh, buf, out, s):
    for k in range(SB):
        so = pl.ds(s * STRIPE + k * B, B)
        pltpu.sync_copy(sh.at[so], buf)
        pltpu.sync_copy(buf, out.at[so])


def _segsum_body(h, srcp2, dstp2, zrow, outA0, outA1,
                 srcv, dstv, r0, r1, s0, s1, A_sh):
    c = lax.axis_index("c")
    s = lax.axis_index("s")
    wid = c * NS + s
    rows = (r0, r1)
    sems = (s0, s1)

    pltpu.sync_copy(zrow, r0)
    _zero_stripe(A_sh, r0, s)
    plsc.subcore_barrier()

    for p in range(CH // PH):
        pbase = wid * CH + p * PH
        pltpu.sync_copy(srcp2.at[pl.ds(pbase, PH)], srcv)
        pltpu.sync_copy(dstp2.at[pl.ds(pbase, PH)], dstv)
        for j in range(NBUF):
            pltpu.async_copy(h.at[srcv.at[j]], rows[j], sems[j])

        def outer(it, carry):
            g = it * NBUF
            for j in range(NBUF):
                cc = g + j
                pltpu.make_async_copy(h.at[srcv.at[cc]], rows[j], sems[j]).wait()
                pltpu.sync_copy(rows[j], A_sh.at[dstv.at[cc]], add=True)

                @pl.when(cc + NBUF < PH)
                def _():
                    pltpu.async_copy(h.at[srcv.at[cc + NBUF]], rows[j], sems[j])
            return carry

        lax.fori_loop(0, PH // NBUF, outer, 0)
    plsc.subcore_barrier()

    @pl.when(c == 0)
    def _():
        _copy_out_stripe(A_sh, r0, outA0, s)

    @pl.when(c == 1)
    def _():
        _copy_out_stripe(A_sh, r0, outA1, s)


_segsum = pl.kernel(
    _segsum_body,
    out_type=[jax.ShapeDtypeStruct((NP, D), _f32),
              jax.ShapeDtypeStruct((NP, D), _f32)],
    mesh=_mesh,
    scratch_types=[
        pltpu.VMEM((PH, B), jnp.int32),
        pltpu.VMEM((PH, B), jnp.int32),
        pltpu.VMEM((B, D), _f32),
        pltpu.VMEM((B, D), _f32),
        pltpu.SemaphoreType.DMA,
        pltpu.SemaphoreType.DMA,
        pltpu.VMEM_SHARED((NP, D), _f32),
    ],
)


_RB = 2000


def _dense_step(a0, a1, d0, d1, w, bb):
    deg = d0[..., 0:1] + d1[..., 0:1]
    acc = jnp.dot(a0 + a1, w[...], preferred_element_type=_f32)
    return jnp.maximum(acc + deg * bb[...], 0.0)


def _conv_body(a0, a1, d0, d1, w, bb, o):
    o[...] = _dense_step(a0[...], a1[...], d0[...], d1[...], w, bb)


def _conv_tc(A0, A1, DG0, DG1, W, b):
    blk = lambda i: (i, 0)
    fixed = lambda i: (0, 0)
    return pl.pallas_call(
        _conv_body,
        grid=(N // _RB,),
        in_specs=[
            pl.BlockSpec((_RB, D), blk),
            pl.BlockSpec((_RB, D), blk),
            pl.BlockSpec((_RB, D), blk),
            pl.BlockSpec((_RB, D), blk),
            pl.BlockSpec((D, D), fixed),
            pl.BlockSpec((1, D), fixed),
        ],
        out_specs=pl.BlockSpec((_RB, D), blk),
        out_shape=jax.ShapeDtypeStruct((N, D), _f32),
    )(A0, A1, DG0, DG1, W, b.reshape(1, D))


def _final_body(a0, a1, d0, d1, w, bb, m0, c0, m1, c1, m2, c2, o):
    h = _dense_step(a0[...], a1[...], d0[...], d1[...], w, bb)
    y = jnp.maximum(jnp.dot(h, m0[...], preferred_element_type=_f32) + c0[...], 0.0)
    y = jnp.maximum(jnp.dot(y, m1[...], preferred_element_type=_f32) + c1[...], 0.0)
    o[...] = jnp.dot(y, m2[...], preferred_element_type=_f32) + c2[...]


def _final_tc(A0, A1, DG0, DG1, W, b, M0, mb0, M1, mb1, M2, mb2):
    blk = lambda i: (i, 0)
    fixed = lambda i: (0, 0)
    return pl.pallas_call(
        _final_body,
        grid=(N // _RB,),
        in_specs=[
            pl.BlockSpec((_RB, D), blk),
            pl.BlockSpec((_RB, D), blk),
            pl.BlockSpec((_RB, D), blk),
            pl.BlockSpec((_RB, D), blk),
            pl.BlockSpec((D, D), fixed),
            pl.BlockSpec((1, D), fixed),
            pl.BlockSpec((D, D), fixed),
            pl.BlockSpec((1, D), fixed),
            pl.BlockSpec((D, D), fixed),
            pl.BlockSpec((1, D), fixed),
            pl.BlockSpec((D, 1), fixed),
            pl.BlockSpec((1, 1), fixed),
        ],
        out_specs=pl.BlockSpec((_RB, 1), blk),
        out_shape=jax.ShapeDtypeStruct((N, 1), _f32),
    )(A0, A1, DG0, DG1, W, b.reshape(1, D),
      M0, mb0.reshape(1, D), M1, mb1.reshape(1, D), M2, mb2.reshape(1, 1))


def kernel(x, edge_index, W0, b0, W1, b1, W2, b2, M0, mb0, M1, mb1, M2, mb2):
    src = edge_index[0]
    dst = edge_index[1]
    pad = EPAD - E
    srcp = jnp.concatenate([src, jnp.zeros((pad,), jnp.int32)]).reshape(NW * CH, B)
    dstp = jnp.concatenate([dst, jnp.full((pad,), N, jnp.int32)]).reshape(NW * CH, B)
    zrow = jnp.zeros((B, D), _f32)
    ones_tab = jnp.ones((N, D), _f32)

    DG0, DG1 = _segsum(ones_tab, srcp, dstp, zrow)
    A0, A1 = _segsum(x, srcp, dstp, zrow)
    h = _conv_tc(A0, A1, DG0, DG1, W0, b0)
    A0, A1 = _segsum(h, srcp, dstp, zrow)
    h = _conv_tc(A0, A1, DG0, DG1, W1, b1)
    A0, A1 = _segsum(h, srcp, dstp, zrow)
    return _final_tc(A0, A1, DG0, DG1, W2, b2, M0, mb0, M1, mb1, M2, mb2)


# speedup vs baseline: 1.1534x; 1.1534x over previous
import jax
import jax.numpy as jnp
from jax import lax
from jax.experimental import pallas as pl
from jax.experimental.pallas import tpu as pltpu
from jax.experimental.pallas import tpu_sc as plsc

N = 10000
D = 128
E = 320000
NC, NS = 2, 16
NW = NC * NS
B = 128
NBUF = 2
CH = ((-(-E // (NW * B)) + NBUF - 1) // NBUF) * NBUF
PH = CH // 2
EPAD = NW * CH * B
SB = 5
STRIPE = SB * B
NP = NS * STRIPE

_mesh = plsc.VectorSubcoreMesh(
    core_axis_name="c", subcore_axis_name="s", num_cores=NC, num_subcores=NS
)
_f32 = jnp.float32


def _zero_stripe(sh, buf, s):
    for k in range(SB):
        pltpu.sync_copy(buf, sh.at[pl.ds(s * STRIPE + k * B, B)])


def _copy_out_stripe(sh, buf, out, s):
    for k in range(SB):
        so = pl.ds(s * STRIPE + k * B, B)
        pltpu.sync_copy(sh.at[so], buf)
        pltpu.sync_copy(buf, out.at[so])


def _segsum_body(h, srcp2, dstp2, zrow, outA0, outA1,
                 srcv, dstv, r0, r1, s0, s1, A_sh):
    c = lax.axis_index("c")
    s = lax.axis_index("s")
    wid = c * NS + s
    pltpu.sync_copy(zrow, r0)
    _zero_stripe(A_sh, r0, s)
    plsc.subcore_barrier()

    for p in range(CH // PH):
        pbase = wid * CH + p * PH
        pltpu.sync_copy(srcp2.at[pl.ds(pbase, PH)], srcv)
        pltpu.sync_copy(dstp2.at[pl.ds(pbase, PH)], dstv)

        def outer(it, carry):
            g = it * NBUF
            d0 = pltpu.async_copy(h.at[srcv.at[g]], r0, s0)
            d1 = pltpu.async_copy(h.at[srcv.at[g + 1]], r1, s1)
            d0.wait()
            pltpu.sync_copy(r0, A_sh.at[dstv.at[g]], add=True)
            d1.wait()
            pltpu.sync_copy(r1, A_sh.at[dstv.at[g + 1]], add=True)
            return carry

        lax.fori_loop(0, PH // NBUF, outer, 0)
    plsc.subcore_barrier()

    @pl.when(c == 0)
    def _():
        _copy_out_stripe(A_sh, r0, outA0, s)

    @pl.when(c == 1)
    def _():
        _copy_out_stripe(A_sh, r0, outA1, s)


_segsum = pl.kernel(
    _segsum_body,
    out_type=[jax.ShapeDtypeStruct((NP, D), _f32),
              jax.ShapeDtypeStruct((NP, D), _f32)],
    mesh=_mesh,
    scratch_types=[
        pltpu.VMEM((PH, B), jnp.int32),
        pltpu.VMEM((PH, B), jnp.int32),
        pltpu.VMEM((B, D), _f32),
        pltpu.VMEM((B, D), _f32),
        pltpu.SemaphoreType.DMA,
        pltpu.SemaphoreType.DMA,
        pltpu.VMEM_SHARED((NP, D), _f32),
    ],
)


def _deg_body(dstp2, zrow, onesr, outD0, outD1, dstv, rows, G_sh):
    c = lax.axis_index("c")
    s = lax.axis_index("s")
    wid = c * NS + s

    pltpu.sync_copy(zrow, rows)
    _zero_stripe(G_sh, rows, s)
    pltpu.sync_copy(dstp2.at[pl.ds(wid * CH, CH)], dstv)
    plsc.subcore_barrier()

    pltpu.sync_copy(onesr, rows)

    def chunk(ci, carry):
        pltpu.sync_copy(rows, G_sh.at[dstv.at[ci]], add=True)
        return carry

    lax.fori_loop(0, CH, chunk, 0)
    plsc.subcore_barrier()

    @pl.when(c == 0)
    def _():
        _copy_out_stripe(G_sh, rows, outD0, s)

    @pl.when(c == 1)
    def _():
        _copy_out_stripe(G_sh, rows, outD1, s)


_deg = pl.kernel(
    _deg_body,
    out_type=[jax.ShapeDtypeStruct((NP, D), _f32),
              jax.ShapeDtypeStruct((NP, D), _f32)],
    mesh=_mesh,
    scratch_types=[
        pltpu.VMEM((CH, B), jnp.int32),
        pltpu.VMEM((B, D), _f32),
        pltpu.VMEM_SHARED((NP, D), _f32),
    ],
)


_RB = 2000


def _dense_step(a0, a1, d0, d1, w, bb):
    deg = d0[..., 0:1] + d1[..., 0:1]
    acc = jnp.dot(a0 + a1, w[...], preferred_element_type=_f32)
    return jnp.maximum(acc + deg * bb[...], 0.0)


def _conv_body(a0, a1, d0, d1, w, bb, o):
    o[...] = _dense_step(a0[...], a1[...], d0[...], d1[...], w, bb)


def _conv_tc(A0, A1, DG0, DG1, W, b):
    blk = lambda i: (i, 0)
    fixed = lambda i: (0, 0)
    return pl.pallas_call(
        _conv_body,
        grid=(N // _RB,),
        in_specs=[
            pl.BlockSpec((_RB, D), blk),
            pl.BlockSpec((_RB, D), blk),
            pl.BlockSpec((_RB, D), blk),
            pl.BlockSpec((_RB, D), blk),
            pl.BlockSpec((D, D), fixed),
            pl.BlockSpec((1, D), fixed),
        ],
        out_specs=pl.BlockSpec((_RB, D), blk),
        out_shape=jax.ShapeDtypeStruct((N, D), _f32),
    )(A0, A1, DG0, DG1, W, b.reshape(1, D))


def _final_body(a0, a1, d0, d1, w, bb, m0, c0, m1, c1, m2, c2, o):
    h = _dense_step(a0[...], a1[...], d0[...], d1[...], w, bb)
    y = jnp.maximum(jnp.dot(h, m0[...], preferred_element_type=_f32) + c0[...], 0.0)
    y = jnp.maximum(jnp.dot(y, m1[...], preferred_element_type=_f32) + c1[...], 0.0)
    o[...] = jnp.dot(y, m2[...], preferred_element_type=_f32) + c2[...]


def _final_tc(A0, A1, DG0, DG1, W, b, M0, mb0, M1, mb1, M2, mb2):
    blk = lambda i: (i, 0)
    fixed = lambda i: (0, 0)
    return pl.pallas_call(
        _final_body,
        grid=(N // _RB,),
        in_specs=[
            pl.BlockSpec((_RB, D), blk),
            pl.BlockSpec((_RB, D), blk),
            pl.BlockSpec((_RB, D), blk),
            pl.BlockSpec((_RB, D), blk),
            pl.BlockSpec((D, D), fixed),
            pl.BlockSpec((1, D), fixed),
            pl.BlockSpec((D, D), fixed),
            pl.BlockSpec((1, D), fixed),
            pl.BlockSpec((D, D), fixed),
            pl.BlockSpec((1, D), fixed),
            pl.BlockSpec((D, 1), fixed),
            pl.BlockSpec((1, 1), fixed),
        ],
        out_specs=pl.BlockSpec((_RB, 1), blk),
        out_shape=jax.ShapeDtypeStruct((N, 1), _f32),
    )(A0, A1, DG0, DG1, W, b.reshape(1, D),
      M0, mb0.reshape(1, D), M1, mb1.reshape(1, D), M2, mb2.reshape(1, 1))


def kernel(x, edge_index, W0, b0, W1, b1, W2, b2, M0, mb0, M1, mb1, M2, mb2):
    src = edge_index[0]
    dst = edge_index[1]
    pad = EPAD - E
    srcp = jnp.concatenate([src, jnp.zeros((pad,), jnp.int32)]).reshape(NW * CH, B)
    dstp = jnp.concatenate([dst, jnp.full((pad,), N, jnp.int32)]).reshape(NW * CH, B)
    zrow = jnp.zeros((B, D), _f32)
    onesr = jnp.ones((B, D), _f32)

    DG0, DG1 = _deg(dstp, zrow, onesr)
    A0, A1 = _segsum(x, srcp, dstp, zrow)
    h = _conv_tc(A0, A1, DG0, DG1, W0, b0)
    A0, A1 = _segsum(h, srcp, dstp, zrow)
    h = _conv_tc(A0, A1, DG0, DG1, W1, b1)
    A0, A1 = _segsum(h, srcp, dstp, zrow)
    return _final_tc(A0, A1, DG0, DG1, W2, b2, M0, mb0, M1, mb1, M2, mb2)


# speedup vs baseline: 1.4386x; 1.2473x over previous
import jax
import jax.numpy as jnp
from jax import lax
from jax.experimental import pallas as pl
from jax.experimental.pallas import tpu as pltpu
from jax.experimental.pallas import tpu_sc as plsc

N = 10000
D = 128
E = 320000
NC, NS = 2, 16
NW = NC * NS
B = 128
NBUF = 2
CH = ((-(-E // (NW * B)) + NBUF - 1) // NBUF) * NBUF
PH = CH // 2
EPAD = NW * CH * B
SB = 5
STRIPE = SB * B
NP = NS * STRIPE

_mesh = plsc.VectorSubcoreMesh(
    core_axis_name="c", subcore_axis_name="s", num_cores=NC, num_subcores=NS
)
_f32 = jnp.float32


def _zero_stripe(sh, buf, s):
    for k in range(SB):
        pltpu.sync_copy(buf, sh.at[pl.ds(s * STRIPE + k * B, B)])


def _copy_out_stripe(sh, buf, out, s):
    for k in range(SB):
        so = pl.ds(s * STRIPE + k * B, B)
        pltpu.sync_copy(sh.at[so], buf)
        pltpu.sync_copy(buf, out.at[so])


def _segsum_body(h, srcp2, dstp, zrow, outA0, outA1,
                 srcv, didx0, didx1, r0, r1, s0, s1, t0, t1, A_sh):
    c = lax.axis_index("c")
    s = lax.axis_index("s")
    wid = c * NS + s
    pltpu.sync_copy(zrow, r0)
    _zero_stripe(A_sh, r0, s)
    plsc.subcore_barrier()

    for p in range(CH // PH):
        pbase = wid * CH + p * PH
        pltpu.sync_copy(srcp2.at[pl.ds(pbase, PH)], srcv)

        def outer(it, carry):
            g = it * NBUF
            eb = (pbase + g) * B
            e0 = pltpu.async_copy(dstp.at[pl.ds(eb, B)], didx0, t0)
            e1 = pltpu.async_copy(dstp.at[pl.ds(eb + B, B)], didx1, t1)
            d0 = pltpu.async_copy(h.at[srcv.at[g]], r0, s0)
            d1 = pltpu.async_copy(h.at[srcv.at[g + 1]], r1, s1)
            e0.wait()
            d0.wait()
            pltpu.sync_copy(r0, A_sh.at[didx0], add=True)
            e1.wait()
            d1.wait()
            pltpu.sync_copy(r1, A_sh.at[didx1], add=True)
            return carry

        lax.fori_loop(0, PH // NBUF, outer, 0)
    plsc.subcore_barrier()

    @pl.when(c == 0)
    def _():
        _copy_out_stripe(A_sh, r0, outA0, s)

    @pl.when(c == 1)
    def _():
        _copy_out_stripe(A_sh, r0, outA1, s)


_segsum = pl.kernel(
    _segsum_body,
    out_type=[jax.ShapeDtypeStruct((NP, D), _f32),
              jax.ShapeDtypeStruct((NP, D), _f32)],
    mesh=_mesh,
    scratch_types=[
        pltpu.VMEM((PH, B), jnp.int32),
        pltpu.VMEM((B,), jnp.int32),
        pltpu.VMEM((B,), jnp.int32),
        pltpu.VMEM((B, D), _f32),
        pltpu.VMEM((B, D), _f32),
        pltpu.SemaphoreType.DMA,
        pltpu.SemaphoreType.DMA,
        pltpu.SemaphoreType.DMA,
        pltpu.SemaphoreType.DMA,
        pltpu.VMEM_SHARED((NP, D), _f32),
    ],
)


_RB = 2000


def _pre_body(hin, w, bb, o):
    o[...] = jnp.dot(hin[...], w[...], preferred_element_type=_f32) + bb[...]


def _pre_tc(hin, W, b):
    blk = lambda i: (i, 0)
    fixed = lambda i: (0, 0)
    return pl.pallas_call(
        _pre_body,
        grid=(N // _RB,),
        in_specs=[
            pl.BlockSpec((_RB, D), blk),
            pl.BlockSpec((D, D), fixed),
            pl.BlockSpec((1, D), fixed),
        ],
        out_specs=pl.BlockSpec((_RB, D), blk),
        out_shape=jax.ShapeDtypeStruct((N, D), _f32),
    )(hin, W, b.reshape(1, D))


def _mid_body(a0, a1, w, bb, o):
    h = jnp.maximum(a0[...] + a1[...], 0.0)
    o[...] = jnp.dot(h, w[...], preferred_element_type=_f32) + bb[...]


def _mid_tc(A0, A1, W, b):
    blk = lambda i: (i, 0)
    fixed = lambda i: (0, 0)
    return pl.pallas_call(
        _mid_body,
        grid=(N // _RB,),
        in_specs=[
            pl.BlockSpec((_RB, D), blk),
            pl.BlockSpec((_RB, D), blk),
            pl.BlockSpec((D, D), fixed),
            pl.BlockSpec((1, D), fixed),
        ],
        out_specs=pl.BlockSpec((_RB, D), blk),
        out_shape=jax.ShapeDtypeStruct((N, D), _f32),
    )(A0, A1, W, b.reshape(1, D))


def _final_body(a0, a1, m0, c0, m1, c1, m2, c2, o):
    h = jnp.maximum(a0[...] + a1[...], 0.0)
    y = jnp.maximum(jnp.dot(h, m0[...], preferred_element_type=_f32) + c0[...], 0.0)
    y = jnp.maximum(jnp.dot(y, m1[...], preferred_element_type=_f32) + c1[...], 0.0)
    o[...] = jnp.dot(y, m2[...], preferred_element_type=_f32) + c2[...]


def _final_tc(A0, A1, M0, mb0, M1, mb1, M2, mb2):
    blk = lambda i: (i, 0)
    fixed = lambda i: (0, 0)
    return pl.pallas_call(
        _final_body,
        grid=(N // _RB,),
        in_specs=[
            pl.BlockSpec((_RB, D), blk),
            pl.BlockSpec((_RB, D), blk),
            pl.BlockSpec((D, D), fixed),
            pl.BlockSpec((1, D), fixed),
            pl.BlockSpec((D, D), fixed),
            pl.BlockSpec((1, D), fixed),
            pl.BlockSpec((D, 1), fixed),
            pl.BlockSpec((1, 1), fixed),
        ],
        out_specs=pl.BlockSpec((_RB, 1), blk),
        out_shape=jax.ShapeDtypeStruct((N, 1), _f32),
    )(A0, A1, M0, mb0.reshape(1, D), M1, mb1.reshape(1, D), M2, mb2.reshape(1, 1))


def kernel(x, edge_index, W0, b0, W1, b1, W2, b2, M0, mb0, M1, mb1, M2, mb2):
    src = edge_index[0]
    dst = edge_index[1]
    pad = EPAD - E
    srcp = jnp.concatenate([src, jnp.zeros((pad,), jnp.int32)]).reshape(NW * CH, B)
    dstp = jnp.concatenate([dst, jnp.full((pad,), N, jnp.int32)])
    zrow = jnp.zeros((B, D), _f32)

    G = _pre_tc(x, W0, b0)
    A0, A1 = _segsum(G, srcp, dstp, zrow)
    G = _mid_tc(A0, A1, W1, b1)
    A0, A1 = _segsum(G, srcp, dstp, zrow)
    G = _mid_tc(A0, A1, W2, b2)
    A0, A1 = _segsum(G, srcp, dstp, zrow)
    return _final_tc(A0, A1, M0, mb0, M1, mb1, M2, mb2)
